# baseline (speedup 1.0000x reference)
import functools

import jax
import jax.numpy as jnp
from jax import lax
from jax.experimental import pallas as pl
from jax.experimental.pallas import tpu as pltpu
from jax.experimental.pallas import tpu_sc as plsc

CHUNK = 128
NBUF = 5


@functools.lru_cache(maxsize=None)
def _emb_lookup(NB, H, V, D):
    info = plsc.get_sparse_core_info()
    NC, NS = info.num_cores, info.num_subcores
    NW = NC * NS
    assert NB % (NW * CHUNK) == 0
    nchunks = H
    assert nchunks % NBUF == 0
    nrounds = nchunks // NBUF
    mesh = plsc.VectorSubcoreMesh(core_axis_name="c", subcore_axis_name="s")

    @functools.partial(
        pl.kernel,
        mesh=mesh,
        out_type=jax.ShapeDtypeStruct((H * NB, D), jnp.float32),
        scratch_types=[
            pltpu.VMEM((H, CHUNK), jnp.int32),
            pltpu.VMEM((NBUF, CHUNK, D), jnp.float32),
            pltpu.VMEM_SHARED((V, D), jnp.float32),
            [pltpu.SemaphoreType.DMA] * NBUF,
            [pltpu.SemaphoreType.DMA] * NBUF,
        ],
        compiler_params=pltpu.CompilerParams(use_tc_tiling_on_sc=True),
    )
    def k(xt_hbm, table_hbm, out_hbm, idx_v, rows_v, table_sp, gsems, osems):
        sid = lax.axis_index("s")
        wid = sid * NC + lax.axis_index("c")

        @pl.when(sid == 0)
        def _():
            pltpu.sync_copy(table_hbm, table_sp)

        pltpu.sync_copy(xt_hbm.at[:, pl.ds(wid * CHUNK, CHUNK)], idx_v)
        plsc.subcore_barrier()

        def out_slice(h):
            return out_hbm.at[pl.ds(h * NB + wid * CHUNK, CHUNK)]

        def gather(h, b):
            return pltpu.async_copy(
                table_sp.at[idx_v.at[h]], rows_v.at[b], gsems[b]
            )

        def write(h, b):
            return pltpu.async_copy(rows_v.at[b], out_slice(h), osems[b])

        for b in range(NBUF):
            gather(b, b)

        def body(i, carry):
            h0 = i * NBUF
            for b in range(NBUF):
                pltpu.make_async_copy(
                    table_sp.at[idx_v.at[h0 + b]], rows_v.at[b], gsems[b]
                ).wait()
                write(h0 + b, b)

            @pl.when(i + 1 < nrounds)
            def _():
                for b in range(NBUF):
                    pltpu.make_async_copy(
                        rows_v.at[b], out_slice(h0 + b), osems[b]
                    ).wait()
                    gather(h0 + NBUF + b, b)

            return carry

        lax.fori_loop(0, nrounds, body, 0)

        for b in range(NBUF):
            pltpu.make_async_copy(
                rows_v.at[b], out_slice(nchunks - NBUF + b), osems[b]
            ).wait()

    return k


def kernel(x, table):
    NB, H = x.shape
    V, D = table.shape
    out = _emb_lookup(NB, H, V, D)(x.T, table)
    return out.reshape(H, NB, D).transpose(1, 0, 2)
